# baseline (speedup 1.0000x reference)
import jax
import jax.numpy as jnp
from jax import lax
from jax.experimental import pallas as pl
from jax.experimental.pallas import tpu as pltpu
from jax.experimental.pallas import tpu_sc as plsc

N = 1048576
M = 4194304
K = 64
R = M // K
NW = 32
QT = N // NW
CB = 128
NCHUNK = QT // CB
L = 16
NG = CB // L


def _count_lt_multi(load, c_vecs, n):
    pos = [jnp.zeros((L,), jnp.int32)] * NG
    s = n // 2
    while s >= 1:
        probes = [load(g, pos[g] + (s - 1)) for g in range(NG)]
        pos = [pos[g] + jnp.where(probes[g] < c_vecs[g], s, 0).astype(jnp.int32)
               for g in range(NG)]
        s //= 2
    probes = [load(g, pos[g]) for g in range(NG)]
    return [pos[g] + (probes[g] < c_vecs[g]).astype(jnp.int32)
            for g in range(NG)]


def _body(cn_hbm, t0_hbm, cs2d_hbm, tail_hbm, out_hbm,
          t0_v, cn_v, out_v, ridx_v, rowsA_v, rowsB_v,
          semC, semA, semB, semO):
    wid = lax.axis_index("s") * 2 + lax.axis_index("c")
    base = wid * QT
    iota = lax.iota(jnp.int32, L)
    zeros16 = jnp.zeros((L,), jnp.int32)
    ones16 = jnp.full((L,), 1, jnp.int32)

    def cn_copy(i, b):
        cbase = base + i * CB
        return pltpu.make_async_copy(
            cn_hbm.at[pl.ds(cbase, CB), :], cn_v.at[b], semC.at[b])

    def gatherA(b):
        return pltpu.make_async_copy(
            cs2d_hbm.at[ridx_v.at[b]], rowsA_v.at[b], semA.at[b])

    def gatherB(b):
        return pltpu.make_async_copy(
            tail_hbm.at[ridx_v.at[b]], rowsB_v.at[b], semB.at[b])

    def out_copy(i, b):
        cbase = base + i * CB
        return pltpu.make_async_copy(
            out_v.at[b], out_hbm.at[pl.ds(cbase, CB)], semO.at[b])

    def coarse(i, b):
        cn_copy(i, b).wait()
        bidx = [iota + g * L for g in range(NG)]
        c_vecs = [plsc.load_gather(cn_v.at[b], [bidx[g], zeros16])
                  for g in range(NG)]
        cts = _count_lt_multi(
            lambda g, p: plsc.load_gather(t0_v, [p]), c_vecs, R)
        for g in range(NG):
            ridx_v[b, pl.ds(g * L, L)] = jnp.maximum(cts[g] - 1, 0)
        gatherA(b).start()
        gatherB(b).start()

    def fine(i, b):
        gatherA(b).wait()
        gatherB(b).wait()

        @pl.when(i >= 4)
        def _():
            out_copy(i, b).wait()
        bidx = [iota + g * L for g in range(NG)]
        c_vecs = [plsc.load_gather(cn_v.at[b], [bidx[g], zeros16])
                  for g in range(NG)]
        os = _count_lt_multi(
            lambda g, p: plsc.load_gather(rowsA_v.at[b], [bidx[g], p]),
            c_vecs, K)
        for g in range(NG):
            b_g = bidx[g]
            r = ridx_v[b, pl.ds(g * L, L)]
            ind = K * r + os[g]
            ind_c = jnp.minimum(ind, M - 1)
            o_c = ind_c - K * r
            wl = plsc.load_gather(rowsA_v.at[b], [b_g, jnp.maximum(o_c - 1, 0)])
            am = plsc.load_gather(rowsA_v.at[b], [b_g, jnp.minimum(o_c, K - 1)])
            bm = plsc.load_gather(rowsB_v.at[b], [b_g, jnp.clip(o_c - K, 0, 15)])
            wm = jnp.where(o_c < K, am, bm)
            jh = o_c + 1
            ah = plsc.load_gather(rowsA_v.at[b], [b_g, jnp.minimum(jh, K - 1)])
            bh = plsc.load_gather(rowsB_v.at[b], [b_g, jnp.clip(jh - K, 0, 15)])
            wh = jnp.where(jh < K, ah, bh)
            zf = jnp.zeros((L,), jnp.float32)
            dlo = jnp.where(ind_c >= 1, wm - wl, zf)
            dhi = jnp.where(ind_c <= M - 2, wh - wm, zf)
            delta = jnp.maximum(dlo, dhi)
            eps = plsc.load_gather(cn_v.at[b], [b_g, ones16])
            out_v[b, pl.ds(g * L, L)] = c_vecs[g] + 0.5 * delta * eps
        out_copy(i, b).start()

        @pl.when(i + 4 < NCHUNK)
        def _():
            cn_copy(i + 4, b).start()

    pltpu.sync_copy(t0_hbm, t0_v)

    for b in range(4):
        cn_copy(b, b).start()
    coarse(0, 0)
    coarse(1, 1)

    def quad(q, carry):
        i0 = 4 * q
        coarse(i0 + 2, 2)
        fine(i0, 0)
        coarse(i0 + 3, 3)
        fine(i0 + 1, 1)

        @pl.when(i0 + 4 < NCHUNK)
        def _():
            coarse(i0 + 4, 0)
        fine(i0 + 2, 2)

        @pl.when(i0 + 5 < NCHUNK)
        def _():
            coarse(i0 + 5, 1)
        fine(i0 + 3, 3)
        return carry

    lax.fori_loop(0, NCHUNK // 4, quad, 0)
    for b in range(4):
        out_copy(NCHUNK - 4 + b, b).wait()


@jax.jit
def kernel(c, cs, deltas, noise_eps):
    del deltas
    mesh = plsc.VectorSubcoreMesh(core_axis_name="c", subcore_axis_name="s")
    run = pl.kernel(
        _body,
        out_type=jax.ShapeDtypeStruct((N,), jnp.float32),
        mesh=mesh,
        scratch_types=[
            pltpu.VMEM((R,), jnp.float32),
            pltpu.VMEM((4, CB, 2), jnp.float32),
            pltpu.VMEM((4, CB), jnp.float32),
            pltpu.VMEM((4, CB), jnp.int32),
            pltpu.VMEM((4, CB, K), jnp.float32),
            pltpu.VMEM((4, CB, 16), jnp.float32),
            pltpu.SemaphoreType.DMA((4,)),
            pltpu.SemaphoreType.DMA((4,)),
            pltpu.SemaphoreType.DMA((4,)),
            pltpu.SemaphoreType.DMA((4,)),
        ],
        compiler_params=pltpu.CompilerParams(
            needs_layout_passes=False, use_tc_tiling_on_sc=False),
    )
    tail_tab = jnp.concatenate([cs[K:], cs[:K]]).reshape(R, K)[:, :16]
    cn = jnp.stack([c.reshape(-1), noise_eps.reshape(-1)], axis=1)
    out = run(cn, cs[::K], cs.reshape(R, K), tail_tab)
    return out.reshape(c.shape)


# speedup vs baseline: 1.1813x; 1.1813x over previous
import jax
import jax.numpy as jnp
from jax import lax
from jax.experimental import pallas as pl
from jax.experimental.pallas import tpu as pltpu
from jax.experimental.pallas import tpu_sc as plsc

N = 1048576
M = 4194304
K = 64
R = M // K
NW = 32
QT = N // NW
CB = 128
NCHUNK = QT // CB
L = 16
NG = CB // L


def _count_lt_multi(load, c_vecs, n):
    pos = [jnp.zeros((L,), jnp.int32)] * NG
    s = n // 2
    while s >= 1:
        probes = [load(g, pos[g] + (s - 1)) for g in range(NG)]
        pos = [pos[g] + jnp.where(probes[g] < c_vecs[g], s, 0).astype(jnp.int32)
               for g in range(NG)]
        s //= 2
    probes = [load(g, pos[g]) for g in range(NG)]
    return [pos[g] + (probes[g] < c_vecs[g]).astype(jnp.int32)
            for g in range(NG)]


def _body(cn_hbm, t0_hbm, cs2d_hbm, tail_hbm, out_hbm,
          t0_v, cn_v, out_v, ridx_v, rowsA_v, rowsB_v,
          semC, semA, semB, semO):
    wid = lax.axis_index("s") * 2 + lax.axis_index("c")
    base = wid * QT
    iota = lax.iota(jnp.int32, L)

    def cn_copy(i, b):
        cbase = 2 * (base + i * CB)
        return pltpu.make_async_copy(
            cn_hbm.at[pl.ds(cbase, 2 * CB)], cn_v.at[b], semC.at[b])

    def gatherA(b):
        return pltpu.make_async_copy(
            cs2d_hbm.at[ridx_v.at[b]], rowsA_v.at[b], semA.at[b])

    def gatherB(b):
        return pltpu.make_async_copy(
            tail_hbm.at[ridx_v.at[b]], rowsB_v.at[b], semB.at[b])

    def out_copy(i, b):
        cbase = base + i * CB
        return pltpu.make_async_copy(
            out_v.at[b], out_hbm.at[pl.ds(cbase, CB)], semO.at[b])

    def coarse(i, b):
        cn_copy(i, b).wait()
        bidx = [iota + g * L for g in range(NG)]
        c_vecs = [plsc.load_gather(cn_v.at[b], [2 * bidx[g]])
                  for g in range(NG)]
        cts = _count_lt_multi(
            lambda g, p: plsc.load_gather(t0_v, [p]), c_vecs, R)
        for g in range(NG):
            ridx_v[b, pl.ds(g * L, L)] = jnp.maximum(cts[g] - 1, 0)
        gatherA(b).start()
        gatherB(b).start()

    def fine(i, b):
        gatherA(b).wait()
        gatherB(b).wait()

        @pl.when(i >= 4)
        def _():
            out_copy(i, b).wait()
        bidx = [iota + g * L for g in range(NG)]
        c_vecs = [plsc.load_gather(cn_v.at[b], [2 * bidx[g]])
                  for g in range(NG)]
        os = _count_lt_multi(
            lambda g, p: plsc.load_gather(rowsA_v.at[b], [bidx[g], p]),
            c_vecs, K)
        for g in range(NG):
            b_g = bidx[g]
            r = ridx_v[b, pl.ds(g * L, L)]
            ind = K * r + os[g]
            ind_c = jnp.minimum(ind, M - 1)
            o_c = ind_c - K * r
            wl = plsc.load_gather(rowsA_v.at[b], [b_g, jnp.maximum(o_c - 1, 0)])
            am = plsc.load_gather(rowsA_v.at[b], [b_g, jnp.minimum(o_c, K - 1)])
            bm = plsc.load_gather(rowsB_v.at[b], [b_g, jnp.clip(o_c - K, 0, 15)])
            wm = jnp.where(o_c < K, am, bm)
            jh = o_c + 1
            ah = plsc.load_gather(rowsA_v.at[b], [b_g, jnp.minimum(jh, K - 1)])
            bh = plsc.load_gather(rowsB_v.at[b], [b_g, jnp.clip(jh - K, 0, 15)])
            wh = jnp.where(jh < K, ah, bh)
            zf = jnp.zeros((L,), jnp.float32)
            dlo = jnp.where(ind_c >= 1, wm - wl, zf)
            dhi = jnp.where(ind_c <= M - 2, wh - wm, zf)
            delta = jnp.maximum(dlo, dhi)
            eps = plsc.load_gather(cn_v.at[b], [2 * b_g + 1])
            out_v[b, pl.ds(g * L, L)] = c_vecs[g] + 0.5 * delta * eps
        out_copy(i, b).start()

        @pl.when(i + 4 < NCHUNK)
        def _():
            cn_copy(i + 4, b).start()

    pltpu.sync_copy(t0_hbm, t0_v)

    for b in range(4):
        cn_copy(b, b).start()
    coarse(0, 0)
    coarse(1, 1)

    def quad(q, carry):
        i0 = 4 * q
        coarse(i0 + 2, 2)
        fine(i0, 0)
        coarse(i0 + 3, 3)
        fine(i0 + 1, 1)

        @pl.when(i0 + 4 < NCHUNK)
        def _():
            coarse(i0 + 4, 0)
        fine(i0 + 2, 2)

        @pl.when(i0 + 5 < NCHUNK)
        def _():
            coarse(i0 + 5, 1)
        fine(i0 + 3, 3)
        return carry

    lax.fori_loop(0, NCHUNK // 4, quad, 0)
    for b in range(4):
        out_copy(NCHUNK - 4 + b, b).wait()


@jax.jit
def kernel(c, cs, deltas, noise_eps):
    del deltas
    mesh = plsc.VectorSubcoreMesh(core_axis_name="c", subcore_axis_name="s")
    run = pl.kernel(
        _body,
        out_type=jax.ShapeDtypeStruct((N,), jnp.float32),
        mesh=mesh,
        scratch_types=[
            pltpu.VMEM((R,), jnp.float32),
            pltpu.VMEM((4, 2 * CB), jnp.float32),
            pltpu.VMEM((4, CB), jnp.float32),
            pltpu.VMEM((4, CB), jnp.int32),
            pltpu.VMEM((4, CB, K), jnp.float32),
            pltpu.VMEM((4, CB, 16), jnp.float32),
            pltpu.SemaphoreType.DMA((4,)),
            pltpu.SemaphoreType.DMA((4,)),
            pltpu.SemaphoreType.DMA((4,)),
            pltpu.SemaphoreType.DMA((4,)),
        ],
        compiler_params=pltpu.CompilerParams(
            needs_layout_passes=False, use_tc_tiling_on_sc=False),
    )
    tail_tab = jnp.concatenate([cs[K:], cs[:K]]).reshape(R, K)[:, :16]
    cn = jnp.stack([c.reshape(-1), noise_eps.reshape(-1)], axis=1).reshape(-1)
    out = run(cn, cs[::K], cs.reshape(R, K), tail_tab)
    return out.reshape(c.shape)
